# Initial kernel scaffold; baseline (speedup 1.0000x reference)
import jax
import jax.numpy as jnp
from jax.experimental import pallas as pl


def kernel(x, table, W, b):
    raise NotImplementedError("write your pallas kernel here")


# speedup vs baseline: 74.2377x; 74.2377x over previous
import functools

import jax
import jax.numpy as jnp
from jax import lax
from jax.experimental import pallas as pl
from jax.experimental.pallas import tpu as pltpu
from jax.experimental.pallas import tpu_sc as plsc

_NC = 2
_NS = 16
_LANES = 16


def _proj_body(table_ref, w_ref, b_ref, v_ref):
    v_ref[...] = (
        jnp.dot(table_ref[...], w_ref[...], preferred_element_type=jnp.float32)
        + b_ref[...]
    )


def _project(table, w_col, b11, row_block):
    vocab, embed = table.shape
    grid = vocab // row_block
    return pl.pallas_call(
        _proj_body,
        grid=(grid,),
        in_specs=[
            pl.BlockSpec((row_block, embed), lambda i: (i, 0)),
            pl.BlockSpec((embed, 1), lambda i: (0, 0)),
            pl.BlockSpec((1, 1), lambda i: (0, 0)),
        ],
        out_specs=pl.BlockSpec((row_block, 1), lambda i: (i, 0)),
        out_shape=jax.ShapeDtypeStruct((vocab, 1), jnp.float32),
    )(table, w_col, b11)


def _make_pool(vocab, batch, seq, chunk_rows, unroll):
    nw = _NC * _NS
    rows_per_w = batch // nw
    n_chunks = rows_per_w // chunk_rows
    chunk_words = chunk_rows * seq
    groups = chunk_rows // _LANES
    mesh = plsc.VectorSubcoreMesh(core_axis_name="c", subcore_axis_name="s")

    @functools.partial(
        pl.kernel,
        out_type=jax.ShapeDtypeStruct((batch,), jnp.float32),
        mesh=mesh,
        compiler_params=pltpu.CompilerParams(needs_layout_passes=False),
        scratch_types=[
            pltpu.VMEM((vocab,), jnp.float32),
            pltpu.VMEM((chunk_words,), jnp.int32),
            pltpu.VMEM((rows_per_w,), jnp.float32),
        ],
    )
    def pool(v_hbm, xf_hbm, out_hbm, v_vmem, idx_vmem, out_vmem):
        wid = lax.axis_index("s") * _NC + lax.axis_index("c")
        base = wid * rows_per_w * seq
        pltpu.sync_copy(v_hbm, v_vmem)
        lane = lax.iota(jnp.int32, _LANES) * seq
        inv = jnp.float32(1.0 / seq)
        for c in range(n_chunks):
            pltpu.sync_copy(
                xf_hbm.at[pl.ds(base + c * chunk_words, chunk_words)], idx_vmem
            )

            def g_body(g, _, c=c):
                gb = g * (_LANES * seq)

                def s_body(t, acc):
                    s0 = t * unroll
                    for u in range(unroll):
                        addr = gb + s0 + u + lane
                        iv = plsc.load_gather(idx_vmem, [addr])
                        acc = acc + plsc.load_gather(v_vmem, [iv])
                    return acc

                acc = lax.fori_loop(
                    0, seq // unroll, s_body, jnp.zeros((_LANES,), jnp.float32)
                )
                out_vmem[pl.ds(c * chunk_rows + g * _LANES, _LANES)] = acc * inv
                return 0

            lax.fori_loop(0, groups, g_body, 0)
        pltpu.sync_copy(out_vmem, out_hbm.at[pl.ds(wid * rows_per_w, rows_per_w)])

    return pool


def kernel(x, table, W, b):
    batch, seq = x.shape
    vocab, embed = table.shape
    v = _project(table, W.reshape(embed, 1), b.reshape(1, 1), row_block=2000)
    v = v.reshape(vocab)
    xf = x.reshape(batch * seq)
    pool = _make_pool(vocab, batch, seq, chunk_rows=128, unroll=8)
    return pool(v, xf)


# speedup vs baseline: 78.2082x; 1.0535x over previous
import functools

import jax
import jax.numpy as jnp
from jax import lax
from jax.experimental import pallas as pl
from jax.experimental.pallas import tpu as pltpu
from jax.experimental.pallas import tpu_sc as plsc

_NC = 2
_NS = 16
_LANES = 16


def _proj_body(table_ref, w_ref, b_ref, v_ref):
    v_ref[...] = (
        jnp.dot(table_ref[...], w_ref[...], preferred_element_type=jnp.float32)
        + b_ref[...]
    )


def _project(table, w_col, b11, row_block):
    vocab, embed = table.shape
    grid = vocab // row_block
    return pl.pallas_call(
        _proj_body,
        grid=(grid,),
        in_specs=[
            pl.BlockSpec((row_block, embed), lambda i: (i, 0)),
            pl.BlockSpec((embed, 1), lambda i: (0, 0)),
            pl.BlockSpec((1, 1), lambda i: (0, 0)),
        ],
        out_specs=pl.BlockSpec((row_block, 1), lambda i: (i, 0)),
        out_shape=jax.ShapeDtypeStruct((vocab, 1), jnp.float32),
    )(table, w_col, b11)


def _make_pool(vocab, batch, seq, chunk_rows, unroll):
    nw = _NC * _NS
    rows_per_w = batch // nw
    n_chunks = rows_per_w // chunk_rows
    groups = chunk_rows // _LANES
    mesh = plsc.VectorSubcoreMesh(core_axis_name="c", subcore_axis_name="s")

    @functools.partial(
        pl.kernel,
        out_type=jax.ShapeDtypeStruct((batch,), jnp.float32),
        mesh=mesh,
        compiler_params=pltpu.CompilerParams(needs_layout_passes=False),
        scratch_types=[
            pltpu.VMEM((vocab,), jnp.float32),
            pltpu.VMEM((2, chunk_rows, seq), jnp.int32),
            pltpu.VMEM((rows_per_w,), jnp.float32),
            pltpu.SemaphoreType.DMA,
            pltpu.SemaphoreType.DMA,
            pltpu.SemaphoreType.DMA,
        ],
    )
    def pool(v_hbm, x_hbm, out_hbm, v_vmem, idx2, out_vmem, vsem, isem0, isem1):
        wid = lax.axis_index("s") * _NC + lax.axis_index("c")
        row0 = wid * rows_per_w
        sems = (isem0, isem1)
        vcopy = pltpu.async_copy(v_hbm, v_vmem, vsem)
        copies = [None, None]
        copies[0] = pltpu.async_copy(
            x_hbm.at[pl.ds(row0, chunk_rows), :], idx2.at[0], sems[0]
        )
        vcopy.wait()
        inv = jnp.float32(1.0 / seq)
        iota = lax.iota(jnp.int32, _LANES)
        for c in range(n_chunks):
            buf = c % 2
            if c + 1 < n_chunks:
                copies[1 - buf] = pltpu.async_copy(
                    x_hbm.at[pl.ds(row0 + (c + 1) * chunk_rows, chunk_rows), :],
                    idx2.at[1 - buf],
                    sems[1 - buf],
                )
            copies[buf].wait()
            idx_ref = idx2.at[buf]
            for g in range(groups):
                rows = g * _LANES + iota

                def s_body(t, acc, idx_ref=idx_ref, rows=rows):
                    s0 = t * unroll
                    for u in range(unroll):
                        cols = jnp.full((_LANES,), 1, jnp.int32) * (s0 + u)
                        iv = plsc.load_gather(idx_ref, [rows, cols])
                        acc = acc + plsc.load_gather(v_vmem, [iv])
                    return acc

                acc = lax.fori_loop(
                    0, seq // unroll, s_body, jnp.zeros((_LANES,), jnp.float32)
                )
                out_vmem[pl.ds(c * chunk_rows + g * _LANES, _LANES)] = acc * inv
        pltpu.sync_copy(out_vmem, out_hbm.at[pl.ds(row0, rows_per_w)])

    return pool


def kernel(x, table, W, b):
    batch, seq = x.shape
    vocab, embed = table.shape
    v = _project(table, W.reshape(embed, 1), b.reshape(1, 1), row_block=10000)
    v = v.reshape(vocab)
    pool = _make_pool(vocab, batch, seq, chunk_rows=32, unroll=8)
    return pool(v, x)


# speedup vs baseline: 220.3822x; 2.8179x over previous
import functools

import jax
import jax.numpy as jnp
from jax import lax
from jax.experimental import pallas as pl
from jax.experimental.pallas import tpu as pltpu
from jax.experimental.pallas import tpu_sc as plsc

_NC = 2
_NS = 16
_LANES = 16


def _proj_body(tableT_ref, w_ref, b_ref, v_ref):
    t = tableT_ref[...]
    w = w_ref[...]
    acc = jnp.sum(t * w, axis=0) + b_ref[0]
    v_ref[...] = acc


def _project(tableT, w_col, b, col_block):
    embed, vocab = tableT.shape
    grid = pl.cdiv(vocab, col_block)
    return pl.pallas_call(
        _proj_body,
        grid=(grid,),
        in_specs=[
            pl.BlockSpec((embed, col_block), lambda i: (0, i)),
            pl.BlockSpec((embed, 1), lambda i: (0, 0)),
            pl.BlockSpec(memory_space=pltpu.SMEM),
        ],
        out_specs=pl.BlockSpec((col_block,), lambda i: (i,)),
        out_shape=jax.ShapeDtypeStruct((vocab,), jnp.float32),
    )(tableT, w_col, b)


def _make_pool(vocab, batch, seq, chunk_cols, unroll):
    nw = _NC * _NS
    cols_per_w = batch // nw
    n_chunks = cols_per_w // chunk_cols
    groups = chunk_cols // _LANES
    mesh = plsc.VectorSubcoreMesh(core_axis_name="c", subcore_axis_name="s")

    @functools.partial(
        pl.kernel,
        out_type=jax.ShapeDtypeStruct((batch,), jnp.float32),
        mesh=mesh,
        compiler_params=pltpu.CompilerParams(needs_layout_passes=False),
        scratch_types=[
            pltpu.VMEM((vocab,), jnp.float32),
            pltpu.VMEM((seq, chunk_cols), jnp.int32),
            pltpu.VMEM((cols_per_w,), jnp.float32),
            pltpu.SemaphoreType.DMA,
        ],
    )
    def pool(v_hbm, xT_hbm, out_hbm, v_vmem, idxc, out_vmem, vsem):
        wid = lax.axis_index("s") * _NC + lax.axis_index("c")
        col0 = wid * cols_per_w
        vcopy = pltpu.async_copy(v_hbm, v_vmem, vsem)
        inv = jnp.float32(1.0 / seq)
        for c in range(n_chunks):
            pltpu.sync_copy(
                xT_hbm.at[:, pl.ds(col0 + c * chunk_cols, chunk_cols)], idxc
            )
            if c == 0:
                vcopy.wait()
            for g in range(groups):
                off = g * _LANES

                def s_body(t, acc, off=off):
                    for u in range(unroll):
                        s = t * unroll + u
                        iv = idxc[s, pl.ds(off, _LANES)]
                        acc = acc + plsc.load_gather(v_vmem, [iv])
                    return acc

                acc = lax.fori_loop(
                    0, seq // unroll, s_body, jnp.zeros((_LANES,), jnp.float32)
                )
                out_vmem[pl.ds(c * chunk_cols + off, _LANES)] = acc * inv
        pltpu.sync_copy(out_vmem, out_hbm.at[pl.ds(col0, cols_per_w)])

    return pool


def kernel(x, table, W, b):
    batch, seq = x.shape
    vocab, embed = table.shape
    v = _project(table.T, W.reshape(embed, 1), b, col_block=16384)
    pool = _make_pool(vocab, batch, seq, chunk_cols=128, unroll=25)
    return pool(v, x.T)


# speedup vs baseline: 250.0881x; 1.1348x over previous
import functools

import jax
import jax.numpy as jnp
from jax import lax
from jax.experimental import pallas as pl
from jax.experimental.pallas import tpu as pltpu
from jax.experimental.pallas import tpu_sc as plsc

_NC = 2
_NS = 16
_LANES = 16


def _proj_body(tableT_ref, w_ref, b_ref, v_ref):
    t = tableT_ref[...]
    w = w_ref[...]
    acc = jnp.sum(t * w, axis=0) + b_ref[0]
    v_ref[...] = acc


def _project(tableT, w_col, b, col_block):
    embed, vocab = tableT.shape
    grid = pl.cdiv(vocab, col_block)
    return pl.pallas_call(
        _proj_body,
        grid=(grid,),
        in_specs=[
            pl.BlockSpec((embed, col_block), lambda i: (0, i)),
            pl.BlockSpec((embed, 1), lambda i: (0, 0)),
            pl.BlockSpec(memory_space=pltpu.SMEM),
        ],
        out_specs=pl.BlockSpec((col_block,), lambda i: (i,)),
        out_shape=jax.ShapeDtypeStruct((vocab,), jnp.float32),
    )(tableT, w_col, b)


def _make_pool(vocab, batch, seq, chunk_cols, unroll):
    nw = _NC * _NS
    cols_per_w = batch // nw
    n_chunks = cols_per_w // chunk_cols
    groups = chunk_cols // _LANES
    halves = (96, seq - 96)
    hoffs = (0, 96)
    n_pieces = 2 * n_chunks
    mesh = plsc.VectorSubcoreMesh(core_axis_name="c", subcore_axis_name="s")

    @functools.partial(
        pl.kernel,
        out_type=jax.ShapeDtypeStruct((batch,), jnp.float32),
        mesh=mesh,
        compiler_params=pltpu.CompilerParams(needs_layout_passes=False),
        scratch_types=[
            pltpu.VMEM((vocab,), jnp.float32),
            pltpu.VMEM((2, max(halves), chunk_cols), jnp.int32),
            pltpu.VMEM((cols_per_w,), jnp.float32),
            pltpu.SemaphoreType.DMA,
            pltpu.SemaphoreType.DMA,
            pltpu.SemaphoreType.DMA,
        ],
    )
    def pool(v_hbm, xT_hbm, out_hbm, v_vmem, idxb, out_vmem, vsem, s0, s1):
        wid = lax.axis_index("s") * _NC + lax.axis_index("c")
        col0 = wid * cols_per_w
        sems = (s0, s1)
        vcopy = pltpu.async_copy(v_hbm, v_vmem, vsem)

        def start(p):
            c, h = p // 2, p % 2
            return pltpu.async_copy(
                xT_hbm.at[
                    pl.ds(hoffs[h], halves[h]),
                    pl.ds(col0 + c * chunk_cols, chunk_cols),
                ],
                idxb.at[p % 2, pl.ds(0, halves[h])],
                sems[p % 2],
            )

        inv = jnp.float32(1.0 / seq)
        copies = [None, None]
        copies[0] = start(0)
        for p in range(n_pieces):
            c, h = p // 2, p % 2
            bi = p % 2
            if p + 1 < n_pieces:
                copies[(p + 1) % 2] = start(p + 1)
            copies[bi].wait()
            if p == 0:
                vcopy.wait()
            buf = idxb.at[bi]
            for g in range(groups):
                off = g * _LANES

                def s_body(t, acc, off=off, buf=buf):
                    for u in range(unroll):
                        s = t * unroll + u
                        iv = buf[s, pl.ds(off, _LANES)]
                        acc = acc + plsc.load_gather(v_vmem, [iv])
                    return acc

                acc = lax.fori_loop(
                    0, halves[h] // unroll, s_body, jnp.zeros((_LANES,), jnp.float32)
                )
                dst = pl.ds(c * chunk_cols + off, _LANES)
                if h == 0:
                    out_vmem[dst] = acc
                else:
                    out_vmem[dst] = (out_vmem[dst] + acc) * inv
        pltpu.sync_copy(out_vmem, out_hbm.at[pl.ds(col0, cols_per_w)])

    return pool


def kernel(x, table, W, b):
    batch, seq = x.shape
    vocab, embed = table.shape
    v = _project(table.T, W.reshape(embed, 1), b, col_block=16384)
    pool = _make_pool(vocab, batch, seq, chunk_cols=128, unroll=8)
    return pool(v, x.T)


# speedup vs baseline: 285.0882x; 1.1400x over previous
import functools

import jax
import jax.numpy as jnp
from jax import lax
from jax.experimental import pallas as pl
from jax.experimental.pallas import tpu as pltpu
from jax.experimental.pallas import tpu_sc as plsc

_NC = 2
_NS = 16
_LANES = 16


def _proj_body(tableT_ref, w_ref, b_ref, v_ref):
    t = tableT_ref[...]
    w = w_ref[...]
    acc = jnp.sum(t * w, axis=0) + b_ref[0]
    v_ref[...] = acc


def _project(tableT, w_col, b, col_block):
    embed, vocab = tableT.shape
    grid = pl.cdiv(vocab, col_block)
    return pl.pallas_call(
        _proj_body,
        grid=(grid,),
        in_specs=[
            pl.BlockSpec((embed, col_block), lambda i: (0, i)),
            pl.BlockSpec((embed, 1), lambda i: (0, 0)),
            pl.BlockSpec(memory_space=pltpu.SMEM),
        ],
        out_specs=pl.BlockSpec((col_block,), lambda i: (i,)),
        out_shape=jax.ShapeDtypeStruct((vocab,), jnp.float32),
    )(tableT, w_col, b)


def _make_pool(vocab, batch, seq, chunk_cols, unroll):
    nw = _NC * _NS
    cols_per_w = batch // nw
    n_chunks = cols_per_w // chunk_cols
    groups = chunk_cols // _LANES
    halves = (56, 48, 48, 48)
    hoffs = (0, 56, 104, 152)
    nh = len(halves)
    n_pieces = nh * n_chunks
    mesh = plsc.VectorSubcoreMesh(core_axis_name="c", subcore_axis_name="s")

    @functools.partial(
        pl.kernel,
        out_type=jax.ShapeDtypeStruct((batch,), jnp.float32),
        mesh=mesh,
        compiler_params=pltpu.CompilerParams(needs_layout_passes=False),
        scratch_types=[
            pltpu.VMEM((vocab,), jnp.float32),
            pltpu.VMEM_SHARED((vocab,), jnp.float32),
            pltpu.VMEM((2, max(halves), chunk_cols), jnp.int32),
            pltpu.VMEM((cols_per_w,), jnp.float32),
            pltpu.SemaphoreType.DMA,
            pltpu.SemaphoreType.DMA,
            pltpu.SemaphoreType.DMA,
        ],
    )
    def pool(v_hbm, xT_hbm, out_hbm, v_vmem, v_shared, idxb, out_vmem, vsem, s0, s1):
        sid = lax.axis_index("s")
        wid = sid * _NC + lax.axis_index("c")
        col0 = wid * cols_per_w
        sems = (s0, s1)

        def start(p):
            c, h = p // nh, p % nh
            return pltpu.async_copy(
                xT_hbm.at[
                    pl.ds(hoffs[h], halves[h]),
                    pl.ds(col0 + c * chunk_cols, chunk_cols),
                ],
                idxb.at[p % 2, pl.ds(0, halves[h])],
                sems[p % 2],
            )

        inv = jnp.float32(1.0 / seq)
        copies = [None, None]
        copies[0] = start(0)

        @pl.when(sid == 0)
        def _stage_v():
            pltpu.sync_copy(v_hbm, v_shared)

        plsc.subcore_barrier()
        vcopy = pltpu.async_copy(v_shared, v_vmem, vsem)
        for p in range(n_pieces):
            c, h = p // nh, p % nh
            bi = p % 2
            if p + 1 < n_pieces:
                copies[(p + 1) % 2] = start(p + 1)
            copies[bi].wait()
            if p == 0:
                vcopy.wait()
            buf = idxb.at[bi]

            def g_body(g, _, buf=buf, c=c, h=h):
                off = pl.multiple_of(g * _LANES, _LANES)

                def s_body(t, acc):
                    for u in range(unroll):
                        s = t * unroll + u
                        iv = buf[s, pl.ds(off, _LANES)]
                        acc = acc + plsc.load_gather(v_vmem, [iv])
                    return acc

                acc = lax.fori_loop(
                    0, halves[h] // unroll, s_body, jnp.zeros((_LANES,), jnp.float32)
                )
                dst = pl.ds(
                    pl.multiple_of(c * chunk_cols + off, _LANES), _LANES
                )
                if h == 0:
                    out_vmem[dst] = acc
                elif h == nh - 1:
                    out_vmem[dst] = (out_vmem[dst] + acc) * inv
                else:
                    out_vmem[dst] = out_vmem[dst] + acc
                return 0

            lax.fori_loop(0, groups, g_body, 0)
        pltpu.sync_copy(out_vmem, out_hbm.at[pl.ds(col0, cols_per_w)])

    return pool


def kernel(x, table, W, b):
    batch, seq = x.shape
    vocab, embed = table.shape
    v = _project(table.T, W.reshape(embed, 1), b, col_block=16384)
    pool = _make_pool(vocab, batch, seq, chunk_cols=128, unroll=8)
    return pool(v, x.T)


# speedup vs baseline: 304.7523x; 1.0690x over previous
import functools

import jax
import jax.numpy as jnp
from jax import lax
from jax.experimental import pallas as pl
from jax.experimental.pallas import tpu as pltpu
from jax.experimental.pallas import tpu_sc as plsc

_NC = 2
_NS = 16
_LANES = 16


def _proj_body(tableT_ref, w_ref, b_ref, v_ref):
    t = tableT_ref[...]
    w = w_ref[...]
    r = lax.dot_general(
        w, t, (((1,), (0,)), ((), ())), preferred_element_type=jnp.float32
    )
    v_ref[...] = jnp.reshape(r, (r.shape[1],)) + b_ref[0]


def _project(tableT, w_row, b, col_block):
    embed, vocab = tableT.shape
    grid = pl.cdiv(vocab, col_block)
    return pl.pallas_call(
        _proj_body,
        grid=(grid,),
        in_specs=[
            pl.BlockSpec((embed, col_block), lambda i: (0, i)),
            pl.BlockSpec((1, embed), lambda i: (0, 0)),
            pl.BlockSpec(memory_space=pltpu.SMEM),
        ],
        out_specs=pl.BlockSpec((col_block,), lambda i: (i,)),
        out_shape=jax.ShapeDtypeStruct((vocab,), jnp.float32),
    )(tableT, w_row, b)


def _make_pool(vocab, batch, seq, chunk_cols, unroll):
    nw = _NC * _NS
    cols_per_w = batch // nw
    n_chunks = cols_per_w // chunk_cols
    groups = chunk_cols // _LANES
    halves = (56, 48, 48, 48)
    hoffs = (0, 56, 104, 152)
    nh = len(halves)
    n_pieces = nh * n_chunks
    mesh = plsc.VectorSubcoreMesh(core_axis_name="c", subcore_axis_name="s")

    @functools.partial(
        pl.kernel,
        out_type=jax.ShapeDtypeStruct((batch,), jnp.float32),
        mesh=mesh,
        compiler_params=pltpu.CompilerParams(needs_layout_passes=False),
        scratch_types=[
            pltpu.VMEM((vocab,), jnp.float32),
            pltpu.VMEM_SHARED((vocab,), jnp.float32),
            pltpu.VMEM((2, max(halves), chunk_cols), jnp.int32),
            pltpu.VMEM((cols_per_w,), jnp.float32),
            pltpu.SemaphoreType.DMA,
            pltpu.SemaphoreType.DMA,
            pltpu.SemaphoreType.DMA,
        ],
    )
    def pool(v_hbm, xT_hbm, out_hbm, v_vmem, v_shared, idxb, out_vmem, vsem, s0, s1):
        sid = lax.axis_index("s")
        wid = sid * _NC + lax.axis_index("c")
        col0 = wid * cols_per_w
        sems = (s0, s1)

        def start(p):
            c, h = p // nh, p % nh
            return pltpu.async_copy(
                xT_hbm.at[
                    pl.ds(hoffs[h], halves[h]),
                    pl.ds(col0 + c * chunk_cols, chunk_cols),
                ],
                idxb.at[p % 2, pl.ds(0, halves[h])],
                sems[p % 2],
            )

        inv = jnp.float32(1.0 / seq)
        copies = [None, None]
        copies[0] = start(0)

        @pl.when(sid == 0)
        def _stage_v():
            pltpu.sync_copy(v_hbm, v_shared)

        plsc.subcore_barrier()
        vcopy = pltpu.async_copy(v_shared, v_vmem, vsem)
        for p in range(n_pieces):
            c, h = p // nh, p % nh
            bi = p % 2
            if p + 1 < n_pieces:
                copies[(p + 1) % 2] = start(p + 1)
            copies[bi].wait()
            if p == 0:
                vcopy.wait()
            buf = idxb.at[bi]

            def g_body(g, _, buf=buf, c=c, h=h):
                off = pl.multiple_of(g * (2 * _LANES), 2 * _LANES)
                zero = jnp.zeros((_LANES,), jnp.float32)

                def s_body(t, accs):
                    a0, a1 = accs
                    for u in range(unroll):
                        s = t * unroll + u
                        iv0 = buf[s, pl.ds(off, _LANES)]
                        iv1 = buf[s, pl.ds(off + _LANES, _LANES)]
                        a0 = a0 + plsc.load_gather(v_vmem, [iv0])
                        a1 = a1 + plsc.load_gather(v_vmem, [iv1])
                    return a0, a1

                a0, a1 = lax.fori_loop(
                    0, halves[h] // unroll, s_body, (zero, zero)
                )
                base = pl.multiple_of(c * chunk_cols + off, 2 * _LANES)
                d0 = pl.ds(base, _LANES)
                d1 = pl.ds(base + _LANES, _LANES)
                if h == 0:
                    out_vmem[d0] = a0
                    out_vmem[d1] = a1
                elif h == nh - 1:
                    out_vmem[d0] = (out_vmem[d0] + a0) * inv
                    out_vmem[d1] = (out_vmem[d1] + a1) * inv
                else:
                    out_vmem[d0] = out_vmem[d0] + a0
                    out_vmem[d1] = out_vmem[d1] + a1
                return 0

            lax.fori_loop(0, groups // 2, g_body, 0)
        pltpu.sync_copy(out_vmem, out_hbm.at[pl.ds(col0, cols_per_w)])

    return pool


def kernel(x, table, W, b):
    batch, seq = x.shape
    vocab, embed = table.shape
    v = _project(table.T, W, b, col_block=50176)
    pool = _make_pool(vocab, batch, seq, chunk_cols=128, unroll=8)
    return pool(v, x.T)
